# baseline (speedup 1.0000x reference)
import jax
import jax.numpy as jnp
from jax.experimental import pallas as pl
from jax.experimental.pallas import tpu as pltpu

N = 256
R = 8
COL_W = 128


def _factor_corner(S):
    i8r = jax.lax.broadcasted_iota(jnp.int32, (1, R, 1), 1)
    i8c = jax.lax.broadcasted_iota(jnp.int32, (1, 1, R), 2)
    for j in range(R):
        piv = S[:, j:j + 1, j:j + 1]
        cmask = (i8c == j).astype(jnp.float32)
        c = jnp.where(i8r > j, S[:, :, j:j + 1] / piv, 0.0)
        rp = jnp.where(i8c > j, S[:, j:j + 1, :], 0.0) + (piv - 1.0) * cmask
        S = S - c * rp
    return S


def _bdot(x, y):
    return jax.lax.dot_general(x, y, (((2,), (1,)), ((0,), (0,))))


def _tri_masks():
    r = jax.lax.broadcasted_iota(jnp.int32, (1, R, R), 1)
    c = jax.lax.broadcasted_iota(jnp.int32, (1, R, R), 2)
    low = (r > c).astype(jnp.float32)
    upp = (r <= c).astype(jnp.float32)
    eye = (r == c).astype(jnp.float32)
    return low, upp, eye


def _inv_unit_lower(Lsl, eye):
    Nl = -Lsl
    N2 = _bdot(Nl, Nl)
    N4 = _bdot(N2, N2)
    return _bdot(_bdot(eye + Nl, eye + N2), eye + N4)


def _inv_upper(Us, d, eye):
    rd = 1.0 / d
    scaled = Us * rd
    Su = scaled - eye
    inv_scaled = _bdot(_bdot(eye - Su, eye + _bdot(Su, Su)),
                       eye + _bdot(_bdot(Su, Su), _bdot(Su, Su)))
    return inv_scaled * jnp.swapaxes(rd, 1, 2)


def _lu_blocked(sref, lo, hi):
    low, upp, eye = _tri_masks()
    Bn = hi - lo
    for t in range(N // R):
        kb = t * R
        coff = (kb // COL_W) * COL_W
        MR, MC = N - kb, N - coff
        kc = kb - coff

        S0 = sref[lo:hi, kb:kb + R, kb:kb + R]
        S = _factor_corner(S0)
        Lsl = S * low
        Us = S * upp
        d = jnp.sum(S * eye, axis=2, keepdims=True)
        invLs = _inv_unit_lower(Lsl, eye)
        invUs = _inv_upper(Us, d, eye)

        A = sref[lo:hi, kb:, coff:]
        if MR > R:
            A21 = sref[lo:hi, kb + R:, kb:kb + R]
            L21 = _bdot(A21, invUs)
            C = jnp.concatenate([Lsl, L21], axis=1)
            A12 = sref[lo:hi, kb:kb + R, :][:, :, kb + R:]
            U12 = _bdot(invLs, A12)
            parts = [Us - eye, U12]
        else:
            C = Lsl
            parts = [Us - eye]
        if kc > 0:
            parts = [jnp.zeros((Bn, R, kc), jnp.float32)] + parts
        Rm = jnp.concatenate(parts, axis=2) if len(parts) > 1 else parts[0]

        sref[lo:hi, kb:, coff:] = A - _bdot(C, Rm)


def _masks_2x2():
    r = jax.lax.broadcasted_iota(jnp.int32, (N, N), 0)
    c = jax.lax.broadcasted_iota(jnp.int32, (N, N), 1)
    def m(i, j):
        return ((r == i) & (c == j)).astype(jnp.float32)
    return m


def _lu_kernel(x_ref, o_ref, s):
    m = _masks_2x2()

    s[0] = x_ref[0]
    s[1] = x_ref[1]
    s[2] = x_ref[2]
    s[3] = x_ref[5]
    s[4] = x_ref[6]
    _lu_blocked(s, 0, 5)
    o_ref[0] = s[0]
    o_ref[1] = s[1]
    o_ref[2] = s[2]
    o_ref[5] = s[3]
    o_ref[6] = s[4]
    o_ref[4] = x_ref[4]

    v8_b0 = s[0, 1:2, 1:2]

    b1, b2, b5, b6 = s[1], s[2], s[3], s[4]
    corr3 = ((b1[1:2, 1:2] + b2[2:3, 2:3]) * m(0, 0)
             + b2[2:3, 3:4] * m(0, 1)
             + b2[3:4, 2:3] * m(1, 0)
             + b2[3:4, 3:4] * m(1, 1))
    corr7 = ((b5[1:2, 1:2] + b6[3:4, 3:4]) * m(0, 0)
             + b6[3:4, 4:5] * m(0, 1)
             + b6[4:5, 3:4] * m(1, 0)
             + b6[4:5, 4:5] * m(1, 1))

    s[0] = x_ref[3] - corr3
    s[1] = x_ref[7] - corr7
    _lu_blocked(s, 0, 2)
    o_ref[3] = s[0]
    o_ref[7] = s[1]

    corr8 = (v8_b0 + s[0, 1:2, 1:2] + s[1, 1:2, 1:2]) * m(0, 0)
    s[0] = x_ref[8] - corr8
    _lu_blocked(s, 0, 1)
    o_ref[8] = s[0]


def kernel(input):
    return pl.pallas_call(
        _lu_kernel,
        out_shape=jax.ShapeDtypeStruct((9, N, N), jnp.float32),
        scratch_shapes=[pltpu.VMEM((5, N, N), jnp.float32)],
    )(input)


# speedup vs baseline: 1.2195x; 1.2195x over previous
import jax
import jax.numpy as jnp
from jax.experimental import pallas as pl
from jax.experimental.pallas import tpu as pltpu

N = 256
R = 8
COL_W = 128


def _factor_corner(S):
    i8r = jax.lax.broadcasted_iota(jnp.int32, (1, R, 1), 1)
    i8c = jax.lax.broadcasted_iota(jnp.int32, (1, 1, R), 2)
    rinvs = []
    for j in range(R):
        piv = S[:, j:j + 1, j:j + 1]
        rinv = 1.0 / piv
        rinvs.append(rinv)
        cmask = (i8c == j).astype(jnp.float32)
        c = jnp.where(i8r > j, S[:, :, j:j + 1] * rinv, 0.0)
        rp = jnp.where(i8c > j, S[:, j:j + 1, :], 0.0) + (piv - 1.0) * cmask
        S = S - c * rp
    return S, jnp.concatenate(rinvs, axis=1)


def _bdot(x, y):
    return jax.lax.dot_general(x, y, (((2,), (1,)), ((0,), (0,))))


def _tri_masks():
    r = jax.lax.broadcasted_iota(jnp.int32, (1, R, R), 1)
    c = jax.lax.broadcasted_iota(jnp.int32, (1, R, R), 2)
    low = (r > c).astype(jnp.float32)
    upp = (r <= c).astype(jnp.float32)
    eye = (r == c).astype(jnp.float32)
    return low, upp, eye


def _inv_unit_lower(Lsl, eye):
    Nl = -Lsl
    N2 = _bdot(Nl, Nl)
    N4 = _bdot(N2, N2)
    return _bdot(_bdot(eye + Nl, eye + N2), eye + N4)


def _inv_upper(Us, rd, eye):
    scaled = Us * rd
    Su = scaled - eye
    Su2 = _bdot(Su, Su)
    inv_scaled = _bdot(_bdot(eye - Su, eye + Su2), eye + _bdot(Su2, Su2))
    return inv_scaled * jnp.swapaxes(rd, 1, 2)


def _lu_blocked(sref, lo, hi):
    low, upp, eye = _tri_masks()
    ROW_W = 64
    for ch in range(N // ROW_W):
        roff = ch * ROW_W
        coff = (roff // COL_W) * COL_W
        MR, MC = N - roff, N - coff
        ec = jax.lax.broadcasted_iota(jnp.int32, (MC, R), 0)
        ei = jax.lax.broadcasted_iota(jnp.int32, (MC, R), 1)
        e2r = jax.lax.broadcasted_iota(jnp.int32, (R, MC), 0)
        e2c = jax.lax.broadcasted_iota(jnp.int32, (R, MC), 1)
        cols = jax.lax.broadcasted_iota(jnp.int32, (1, 1, MC), 2)

        def body(t, carry, roff=roff, coff=coff, MR=MR, MC=MC,
                 ec=ec, ei=ei, e2r=e2r, e2c=e2c, cols=cols):
            Bn = hi - lo
            kr = t * R
            kc = (roff - coff) + t * R
            A = sref[lo:hi, roff:, coff:]
            Rw = sref[lo:hi, pl.ds(roff + t * R, R), :][:, :, coff:]
            E = (ec == ei + kc).astype(jnp.float32)
            P = jax.lax.dot_general(A, E, (((2,), (0,)), ((), ())))
            S0 = jax.lax.dot_general(Rw, E, (((2,), (0,)), ((), ())))

            S, rd = _factor_corner(S0)
            Lsl = S * low
            Us = S * upp
            invLs = _inv_unit_lower(Lsl, eye)
            invUs = _inv_upper(Us, rd, eye)

            rows = jax.lax.broadcasted_iota(jnp.int32, (1, MR, 1), 1)
            L21 = _bdot(P, invUs)
            gr = jax.lax.broadcasted_iota(jnp.int32, (MR, R), 0)
            gp = jax.lax.broadcasted_iota(jnp.int32, (MR, R), 1)
            G = jnp.broadcast_to((gr == gp + kr).astype(jnp.float32),
                                 (Bn, MR, R))
            C = _bdot(G, Lsl) + jnp.where(rows >= kr + R, L21, 0.0)
            U12f = _bdot(invLs, Rw)
            E2 = (e2c == e2r + kc).astype(jnp.float32)
            Rm = (jnp.where(cols >= kc + R, U12f, 0.0)
                  + jax.lax.dot_general(Us - eye, E2,
                                        (((2,), (0,)), ((), ()))))

            sref[lo:hi, roff:, coff:] = A - _bdot(C, Rm)
            return carry

        jax.lax.fori_loop(0, ROW_W // R, body, 0)


def _masks_2x2():
    r = jax.lax.broadcasted_iota(jnp.int32, (N, N), 0)
    c = jax.lax.broadcasted_iota(jnp.int32, (N, N), 1)
    def m(i, j):
        return ((r == i) & (c == j)).astype(jnp.float32)
    return m


def _lu_kernel(x_ref, o_ref, s):
    m = _masks_2x2()

    s[0] = x_ref[0]
    s[1] = x_ref[1]
    s[2] = x_ref[2]
    s[3] = x_ref[5]
    s[4] = x_ref[6]
    _lu_blocked(s, 0, 5)
    o_ref[0] = s[0]
    o_ref[1] = s[1]
    o_ref[2] = s[2]
    o_ref[5] = s[3]
    o_ref[6] = s[4]
    o_ref[4] = x_ref[4]

    v8_b0 = s[0, 1:2, 1:2]

    b1, b2, b5, b6 = s[1], s[2], s[3], s[4]
    corr3 = ((b1[1:2, 1:2] + b2[2:3, 2:3]) * m(0, 0)
             + b2[2:3, 3:4] * m(0, 1)
             + b2[3:4, 2:3] * m(1, 0)
             + b2[3:4, 3:4] * m(1, 1))
    corr7 = ((b5[1:2, 1:2] + b6[3:4, 3:4]) * m(0, 0)
             + b6[3:4, 4:5] * m(0, 1)
             + b6[4:5, 3:4] * m(1, 0)
             + b6[4:5, 4:5] * m(1, 1))

    s[0] = x_ref[3] - corr3
    s[1] = x_ref[7] - corr7
    _lu_blocked(s, 0, 2)
    o_ref[3] = s[0]
    o_ref[7] = s[1]

    corr8 = (v8_b0 + s[0, 1:2, 1:2] + s[1, 1:2, 1:2]) * m(0, 0)
    s[0] = x_ref[8] - corr8
    _lu_blocked(s, 0, 1)
    o_ref[8] = s[0]


def kernel(input):
    return pl.pallas_call(
        _lu_kernel,
        out_shape=jax.ShapeDtypeStruct((9, N, N), jnp.float32),
        scratch_shapes=[pltpu.VMEM((5, N, N), jnp.float32)],
    )(input)


# speedup vs baseline: 1.5951x; 1.3080x over previous
import jax
import jax.numpy as jnp
from jax.experimental import pallas as pl
from jax.experimental.pallas import tpu as pltpu

N = 256
R = 16
COL_W = 128


def _factor_corner(S):
    i8r = jax.lax.broadcasted_iota(jnp.int32, (1, R, 1), 1)
    i8c = jax.lax.broadcasted_iota(jnp.int32, (1, 1, R), 2)
    rinvs = []
    for j in range(R):
        piv = S[:, j:j + 1, j:j + 1]
        rinv = 1.0 / piv
        rinvs.append(rinv)
        cmask = (i8c == j).astype(jnp.float32)
        c = jnp.where(i8r > j, S[:, :, j:j + 1] * rinv, 0.0)
        rp = jnp.where(i8c > j, S[:, j:j + 1, :], 0.0) + (piv - 1.0) * cmask
        S = S - c * rp
    return S, jnp.concatenate(rinvs, axis=1)


def _bdot(x, y):
    return jax.lax.dot_general(x, y, (((2,), (1,)), ((0,), (0,))))


def _tri_masks():
    r = jax.lax.broadcasted_iota(jnp.int32, (1, R, R), 1)
    c = jax.lax.broadcasted_iota(jnp.int32, (1, R, R), 2)
    low = (r > c).astype(jnp.float32)
    upp = (r <= c).astype(jnp.float32)
    eye = (r == c).astype(jnp.float32)
    return low, upp, eye


def _neumann_inv(T, eye):
    P = -T
    out = eye + P
    k = 2
    while k < R:
        P = _bdot(P, P)
        out = _bdot(out, eye + P)
        k *= 2
    return out


def _inv_unit_lower(Lsl, eye):
    return _neumann_inv(Lsl, eye)


def _inv_upper(Us, rd, eye):
    scaled = Us * rd
    inv_scaled = _neumann_inv(scaled - eye, eye)
    return inv_scaled * jnp.swapaxes(rd, 1, 2)


def _lu_blocked(sref, lo, hi):
    low, upp, eye = _tri_masks()
    ROW_W = 64
    for ch in range(N // ROW_W):
        roff = ch * ROW_W
        coff = (roff // COL_W) * COL_W
        MR, MC = N - roff, N - coff
        ec = jax.lax.broadcasted_iota(jnp.int32, (MC, R), 0)
        ei = jax.lax.broadcasted_iota(jnp.int32, (MC, R), 1)
        e2r = jax.lax.broadcasted_iota(jnp.int32, (R, MC), 0)
        e2c = jax.lax.broadcasted_iota(jnp.int32, (R, MC), 1)
        cols = jax.lax.broadcasted_iota(jnp.int32, (1, 1, MC), 2)

        def body(t, carry, roff=roff, coff=coff, MR=MR, MC=MC,
                 ec=ec, ei=ei, e2r=e2r, e2c=e2c, cols=cols):
            Bn = hi - lo
            kr = t * R
            kc = (roff - coff) + t * R
            A = sref[lo:hi, roff:, coff:]
            Rw = sref[lo:hi, pl.ds(roff + t * R, R), :][:, :, coff:]
            E = (ec == ei + kc).astype(jnp.float32)
            P = jax.lax.dot_general(A, E, (((2,), (0,)), ((), ())))
            S0 = jax.lax.dot_general(Rw, E, (((2,), (0,)), ((), ())))

            S, rd = _factor_corner(S0)
            Lsl = S * low
            Us = S * upp
            invLs = _inv_unit_lower(Lsl, eye)
            invUs = _inv_upper(Us, rd, eye)

            rows = jax.lax.broadcasted_iota(jnp.int32, (1, MR, 1), 1)
            L21 = _bdot(P, invUs)
            gr = jax.lax.broadcasted_iota(jnp.int32, (MR, R), 0)
            gp = jax.lax.broadcasted_iota(jnp.int32, (MR, R), 1)
            G = jnp.broadcast_to((gr == gp + kr).astype(jnp.float32),
                                 (Bn, MR, R))
            C = _bdot(G, Lsl) + jnp.where(rows >= kr + R, L21, 0.0)
            U12f = _bdot(invLs, Rw)
            E2 = (e2c == e2r + kc).astype(jnp.float32)
            Rm = (jnp.where(cols >= kc + R, U12f, 0.0)
                  + jax.lax.dot_general(Us - eye, E2,
                                        (((2,), (0,)), ((), ()))))

            sref[lo:hi, roff:, coff:] = A - _bdot(C, Rm)
            return carry

        jax.lax.fori_loop(0, ROW_W // R, body, 0)


def _masks_2x2():
    r = jax.lax.broadcasted_iota(jnp.int32, (N, N), 0)
    c = jax.lax.broadcasted_iota(jnp.int32, (N, N), 1)
    def m(i, j):
        return ((r == i) & (c == j)).astype(jnp.float32)
    return m


def _lu_kernel(x_ref, o_ref, s):
    m = _masks_2x2()

    s[0] = x_ref[0]
    s[1] = x_ref[1]
    s[2] = x_ref[2]
    s[3] = x_ref[5]
    s[4] = x_ref[6]
    _lu_blocked(s, 0, 5)
    o_ref[0] = s[0]
    o_ref[1] = s[1]
    o_ref[2] = s[2]
    o_ref[5] = s[3]
    o_ref[6] = s[4]
    o_ref[4] = x_ref[4]

    v8_b0 = s[0, 1:2, 1:2]

    b1, b2, b5, b6 = s[1], s[2], s[3], s[4]
    corr3 = ((b1[1:2, 1:2] + b2[2:3, 2:3]) * m(0, 0)
             + b2[2:3, 3:4] * m(0, 1)
             + b2[3:4, 2:3] * m(1, 0)
             + b2[3:4, 3:4] * m(1, 1))
    corr7 = ((b5[1:2, 1:2] + b6[3:4, 3:4]) * m(0, 0)
             + b6[3:4, 4:5] * m(0, 1)
             + b6[4:5, 3:4] * m(1, 0)
             + b6[4:5, 4:5] * m(1, 1))

    s[0] = x_ref[3] - corr3
    s[1] = x_ref[7] - corr7
    _lu_blocked(s, 0, 2)
    o_ref[3] = s[0]
    o_ref[7] = s[1]

    corr8 = (v8_b0 + s[0, 1:2, 1:2] + s[1, 1:2, 1:2]) * m(0, 0)
    s[0] = x_ref[8] - corr8
    _lu_blocked(s, 0, 1)
    o_ref[8] = s[0]


def kernel(input):
    return pl.pallas_call(
        _lu_kernel,
        out_shape=jax.ShapeDtypeStruct((9, N, N), jnp.float32),
        scratch_shapes=[pltpu.VMEM((5, N, N), jnp.float32)],
    )(input)


# speedup vs baseline: 1.9839x; 1.2437x over previous
import jax
import jax.numpy as jnp
from jax.experimental import pallas as pl
from jax.experimental.pallas import tpu as pltpu

N = 256
R = 32
COL_W = 128


def _factor_corner(S):
    i8r = jax.lax.broadcasted_iota(jnp.int32, (1, R, 1), 1)
    i8c = jax.lax.broadcasted_iota(jnp.int32, (1, 1, R), 2)
    rinvs = []
    for j in range(R):
        piv = S[:, j:j + 1, j:j + 1]
        rinv = 1.0 / piv
        rinvs.append(rinv)
        cmask = (i8c == j).astype(jnp.float32)
        c = jnp.where(i8r > j, S[:, :, j:j + 1] * rinv, 0.0)
        rp = jnp.where(i8c > j, S[:, j:j + 1, :], 0.0) + (piv - 1.0) * cmask
        S = S - c * rp
    return S, jnp.concatenate(rinvs, axis=1)


def _bdot(x, y):
    return jax.lax.dot_general(x, y, (((2,), (1,)), ((0,), (0,))))


def _tri_masks():
    r = jax.lax.broadcasted_iota(jnp.int32, (1, R, R), 1)
    c = jax.lax.broadcasted_iota(jnp.int32, (1, R, R), 2)
    low = (r > c).astype(jnp.float32)
    upp = (r <= c).astype(jnp.float32)
    eye = (r == c).astype(jnp.float32)
    return low, upp, eye


def _neumann_inv(T, eye):
    P = -T
    out = eye + P
    k = 2
    while k < R:
        P = _bdot(P, P)
        out = _bdot(out, eye + P)
        k *= 2
    return out


def _inv_unit_lower(Lsl, eye):
    return _neumann_inv(Lsl, eye)


def _inv_upper(Us, rd, eye):
    scaled = Us * rd
    inv_scaled = _neumann_inv(scaled - eye, eye)
    return inv_scaled * jnp.swapaxes(rd, 1, 2)


def _lu_blocked(sref, lo, hi):
    low, upp, eye = _tri_masks()
    ROW_W = 64
    for ch in range(N // ROW_W):
        roff = ch * ROW_W
        coff = (roff // COL_W) * COL_W
        MR, MC = N - roff, N - coff
        ec = jax.lax.broadcasted_iota(jnp.int32, (MC, R), 0)
        ei = jax.lax.broadcasted_iota(jnp.int32, (MC, R), 1)
        e2r = jax.lax.broadcasted_iota(jnp.int32, (R, MC), 0)
        e2c = jax.lax.broadcasted_iota(jnp.int32, (R, MC), 1)
        cols = jax.lax.broadcasted_iota(jnp.int32, (1, 1, MC), 2)

        def body(t, carry, roff=roff, coff=coff, MR=MR, MC=MC,
                 ec=ec, ei=ei, e2r=e2r, e2c=e2c, cols=cols):
            Bn = hi - lo
            kr = t * R
            kc = (roff - coff) + t * R
            A = sref[lo:hi, roff:, coff:]
            Rw = sref[lo:hi, pl.ds(roff + t * R, R), :][:, :, coff:]
            E = (ec == ei + kc).astype(jnp.float32)
            P = jax.lax.dot_general(A, E, (((2,), (0,)), ((), ())))
            S0 = jax.lax.dot_general(Rw, E, (((2,), (0,)), ((), ())))

            S, rd = _factor_corner(S0)
            Lsl = S * low
            Us = S * upp
            invLs = _inv_unit_lower(Lsl, eye)
            invUs = _inv_upper(Us, rd, eye)

            rows = jax.lax.broadcasted_iota(jnp.int32, (1, MR, 1), 1)
            L21 = _bdot(P, invUs)
            gr = jax.lax.broadcasted_iota(jnp.int32, (MR, R), 0)
            gp = jax.lax.broadcasted_iota(jnp.int32, (MR, R), 1)
            G = jnp.broadcast_to((gr == gp + kr).astype(jnp.float32),
                                 (Bn, MR, R))
            C = _bdot(G, Lsl) + jnp.where(rows >= kr + R, L21, 0.0)
            U12f = _bdot(invLs, Rw)
            E2 = (e2c == e2r + kc).astype(jnp.float32)
            Rm = (jnp.where(cols >= kc + R, U12f, 0.0)
                  + jax.lax.dot_general(Us - eye, E2,
                                        (((2,), (0,)), ((), ()))))

            sref[lo:hi, roff:, coff:] = A - _bdot(C, Rm)
            return carry

        jax.lax.fori_loop(0, ROW_W // R, body, 0)


def _masks_2x2():
    r = jax.lax.broadcasted_iota(jnp.int32, (N, N), 0)
    c = jax.lax.broadcasted_iota(jnp.int32, (N, N), 1)
    def m(i, j):
        return ((r == i) & (c == j)).astype(jnp.float32)
    return m


def _lu_kernel(x_ref, o_ref, s):
    m = _masks_2x2()

    s[0] = x_ref[0]
    s[1] = x_ref[1]
    s[2] = x_ref[2]
    s[3] = x_ref[5]
    s[4] = x_ref[6]
    _lu_blocked(s, 0, 5)
    o_ref[0] = s[0]
    o_ref[1] = s[1]
    o_ref[2] = s[2]
    o_ref[5] = s[3]
    o_ref[6] = s[4]
    o_ref[4] = x_ref[4]

    v8_b0 = s[0, 1:2, 1:2]

    b1, b2, b5, b6 = s[1], s[2], s[3], s[4]
    corr3 = ((b1[1:2, 1:2] + b2[2:3, 2:3]) * m(0, 0)
             + b2[2:3, 3:4] * m(0, 1)
             + b2[3:4, 2:3] * m(1, 0)
             + b2[3:4, 3:4] * m(1, 1))
    corr7 = ((b5[1:2, 1:2] + b6[3:4, 3:4]) * m(0, 0)
             + b6[3:4, 4:5] * m(0, 1)
             + b6[4:5, 3:4] * m(1, 0)
             + b6[4:5, 4:5] * m(1, 1))

    s[0] = x_ref[3] - corr3
    s[1] = x_ref[7] - corr7
    _lu_blocked(s, 0, 2)
    o_ref[3] = s[0]
    o_ref[7] = s[1]

    corr8 = (v8_b0 + s[0, 1:2, 1:2] + s[1, 1:2, 1:2]) * m(0, 0)
    s[0] = x_ref[8] - corr8
    _lu_blocked(s, 0, 1)
    o_ref[8] = s[0]


def kernel(input):
    return pl.pallas_call(
        _lu_kernel,
        out_shape=jax.ShapeDtypeStruct((9, N, N), jnp.float32),
        scratch_shapes=[pltpu.VMEM((5, N, N), jnp.float32)],
    )(input)


# speedup vs baseline: 2.1444x; 1.0809x over previous
import jax
import jax.numpy as jnp
from jax.experimental import pallas as pl
from jax.experimental.pallas import tpu as pltpu

N = 256
R = 64
COL_W = 128


def _factor_corner(S):
    i8r = jax.lax.broadcasted_iota(jnp.int32, (1, R, 1), 1)
    i8c = jax.lax.broadcasted_iota(jnp.int32, (1, 1, R), 2)
    rinvs = []
    for j in range(R):
        piv = S[:, j:j + 1, j:j + 1]
        rinv = 1.0 / piv
        rinvs.append(rinv)
        cmask = (i8c == j).astype(jnp.float32)
        c = jnp.where(i8r > j, S[:, :, j:j + 1] * rinv, 0.0)
        rp = jnp.where(i8c > j, S[:, j:j + 1, :], 0.0) + (piv - 1.0) * cmask
        S = S - c * rp
    return S, jnp.concatenate(rinvs, axis=1)


def _bdot(x, y):
    return jax.lax.dot_general(x, y, (((2,), (1,)), ((0,), (0,))))


def _tri_masks():
    r = jax.lax.broadcasted_iota(jnp.int32, (1, R, R), 1)
    c = jax.lax.broadcasted_iota(jnp.int32, (1, R, R), 2)
    low = (r > c).astype(jnp.float32)
    upp = (r <= c).astype(jnp.float32)
    eye = (r == c).astype(jnp.float32)
    return low, upp, eye


def _neumann_inv(T, eye):
    P = -T
    out = eye + P
    k = 2
    while k < R:
        P = _bdot(P, P)
        out = _bdot(out, eye + P)
        k *= 2
    return out


def _inv_unit_lower(Lsl, eye):
    return _neumann_inv(Lsl, eye)


def _inv_upper(Us, rd, eye):
    scaled = Us * rd
    inv_scaled = _neumann_inv(scaled - eye, eye)
    return inv_scaled * jnp.swapaxes(rd, 1, 2)


def _lu_blocked(sref, lo, hi):
    low, upp, eye = _tri_masks()
    ROW_W = 64
    for ch in range(N // ROW_W):
        roff = ch * ROW_W
        coff = (roff // COL_W) * COL_W
        MR, MC = N - roff, N - coff
        ec = jax.lax.broadcasted_iota(jnp.int32, (MC, R), 0)
        ei = jax.lax.broadcasted_iota(jnp.int32, (MC, R), 1)
        e2r = jax.lax.broadcasted_iota(jnp.int32, (R, MC), 0)
        e2c = jax.lax.broadcasted_iota(jnp.int32, (R, MC), 1)
        cols = jax.lax.broadcasted_iota(jnp.int32, (1, 1, MC), 2)

        def body(t, carry, roff=roff, coff=coff, MR=MR, MC=MC,
                 ec=ec, ei=ei, e2r=e2r, e2c=e2c, cols=cols):
            Bn = hi - lo
            kr = t * R
            kc = (roff - coff) + t * R
            A = sref[lo:hi, roff:, coff:]
            Rw = sref[lo:hi, pl.ds(roff + t * R, R), :][:, :, coff:]
            E = (ec == ei + kc).astype(jnp.float32)
            P = jax.lax.dot_general(A, E, (((2,), (0,)), ((), ())))
            S0 = jax.lax.dot_general(Rw, E, (((2,), (0,)), ((), ())))

            S, rd = _factor_corner(S0)
            Lsl = S * low
            Us = S * upp
            invLs = _inv_unit_lower(Lsl, eye)
            invUs = _inv_upper(Us, rd, eye)

            rows = jax.lax.broadcasted_iota(jnp.int32, (1, MR, 1), 1)
            L21 = _bdot(P, invUs)
            gr = jax.lax.broadcasted_iota(jnp.int32, (MR, R), 0)
            gp = jax.lax.broadcasted_iota(jnp.int32, (MR, R), 1)
            G = jnp.broadcast_to((gr == gp + kr).astype(jnp.float32),
                                 (Bn, MR, R))
            C = _bdot(G, Lsl) + jnp.where(rows >= kr + R, L21, 0.0)
            U12f = _bdot(invLs, Rw)
            E2 = (e2c == e2r + kc).astype(jnp.float32)
            Rm = (jnp.where(cols >= kc + R, U12f, 0.0)
                  + jax.lax.dot_general(Us - eye, E2,
                                        (((2,), (0,)), ((), ()))))

            sref[lo:hi, roff:, coff:] = A - _bdot(C, Rm)
            return carry

        jax.lax.fori_loop(0, ROW_W // R, body, 0)


def _masks_2x2():
    r = jax.lax.broadcasted_iota(jnp.int32, (N, N), 0)
    c = jax.lax.broadcasted_iota(jnp.int32, (N, N), 1)
    def m(i, j):
        return ((r == i) & (c == j)).astype(jnp.float32)
    return m


def _lu_kernel(x_ref, o_ref, s):
    m = _masks_2x2()

    s[0] = x_ref[0]
    s[1] = x_ref[1]
    s[2] = x_ref[2]
    s[3] = x_ref[5]
    s[4] = x_ref[6]
    _lu_blocked(s, 0, 5)
    o_ref[0] = s[0]
    o_ref[1] = s[1]
    o_ref[2] = s[2]
    o_ref[5] = s[3]
    o_ref[6] = s[4]
    o_ref[4] = x_ref[4]

    v8_b0 = s[0, 1:2, 1:2]

    b1, b2, b5, b6 = s[1], s[2], s[3], s[4]
    corr3 = ((b1[1:2, 1:2] + b2[2:3, 2:3]) * m(0, 0)
             + b2[2:3, 3:4] * m(0, 1)
             + b2[3:4, 2:3] * m(1, 0)
             + b2[3:4, 3:4] * m(1, 1))
    corr7 = ((b5[1:2, 1:2] + b6[3:4, 3:4]) * m(0, 0)
             + b6[3:4, 4:5] * m(0, 1)
             + b6[4:5, 3:4] * m(1, 0)
             + b6[4:5, 4:5] * m(1, 1))

    s[0] = x_ref[3] - corr3
    s[1] = x_ref[7] - corr7
    _lu_blocked(s, 0, 2)
    o_ref[3] = s[0]
    o_ref[7] = s[1]

    corr8 = (v8_b0 + s[0, 1:2, 1:2] + s[1, 1:2, 1:2]) * m(0, 0)
    s[0] = x_ref[8] - corr8
    _lu_blocked(s, 0, 1)
    o_ref[8] = s[0]


def kernel(input):
    return pl.pallas_call(
        _lu_kernel,
        out_shape=jax.ShapeDtypeStruct((9, N, N), jnp.float32),
        scratch_shapes=[pltpu.VMEM((5, N, N), jnp.float32)],
    )(input)
